# baseline (device time: 22888 ns/iter reference)
import jax
import jax.numpy as jnp
from jax import lax
from jax.experimental import pallas as pl
from jax.experimental.pallas import tpu as pltpu

N_DEV = 4
B, SQ, SKV, HQ_TOTAL, DH = 2, 128, 128, 16, 64
H_LOC = HQ_TOTAL // N_DEV
CHUNK = H_LOC * DH
ROWS = B * SQ


def kernel(x, Wq, K_ext, V_ext, Wo):
    def body(x_ref, wq_ref, k_ref, v_ref, wo_ref, out_ref,
             comm_ref, send_sems, recv_sems):
        my = lax.axis_index("i")
        left = lax.rem(my + N_DEV - 1, N_DEV)
        right = lax.rem(my + 1, N_DEV)

        barrier_sem = pltpu.get_barrier_semaphore()
        for nbr in (left, right):
            pl.semaphore_signal(
                barrier_sem, inc=1,
                device_id=(nbr,), device_id_type=pl.DeviceIdType.MESH,
            )
        pl.semaphore_wait(barrier_sem, 2)

        x2d = x_ref[...].reshape(ROWS, 512).astype(jnp.bfloat16)
        wq_slice = wq_ref[:, pl.ds(my * CHUNK, CHUNK)].astype(jnp.bfloat16)
        q2d = lax.dot_general(
            x2d, wq_slice, (((1,), (0,)), ((), ())),
            preferred_element_type=jnp.float32,
        ).astype(jnp.bfloat16)

        for b in range(B):
            for h in range(H_LOC):
                qb = q2d[b * SQ:(b + 1) * SQ, h * DH:(h + 1) * DH]
                kb = k_ref[b, :, h, :].astype(jnp.bfloat16)
                vb = v_ref[b, :, h, :].astype(jnp.bfloat16)
                scores = lax.dot_general(
                    qb, kb, (((1,), (1,)), ((), ())),
                    preferred_element_type=jnp.float32,
                ) * 0.125
                m = jnp.max(scores, axis=-1, keepdims=True)
                w = jnp.exp(scores - m)
                w = w / jnp.sum(w, axis=-1, keepdims=True)
                ctx = lax.dot_general(
                    w.astype(jnp.bfloat16), vb, (((1,), (0,)), ((), ())),
                    preferred_element_type=jnp.float32,
                )
                comm_ref[0, b * SQ:(b + 1) * SQ, h * DH:(h + 1) * DH] = (
                    ctx.astype(jnp.bfloat16)
                )

        acc = lax.dot_general(
            comm_ref[0], wo_ref[pl.ds(my * CHUNK, CHUNK), :].astype(jnp.bfloat16),
            (((1,), (0,)), ((), ())),
            preferred_element_type=jnp.float32,
        )

        for hop in range(N_DEV - 1):
            send_slot = hop % 2
            recv_slot = (hop + 1) % 2
            rdma = pltpu.make_async_remote_copy(
                src_ref=comm_ref.at[send_slot],
                dst_ref=comm_ref.at[recv_slot],
                send_sem=send_sems.at[send_slot],
                recv_sem=recv_sems.at[recv_slot],
                device_id=(right,),
                device_id_type=pl.DeviceIdType.MESH,
            )
            rdma.start()
            rdma.wait()

            origin = lax.rem(my + N_DEV - 1 - hop, N_DEV)
            acc += lax.dot_general(
                comm_ref[recv_slot],
                wo_ref[pl.ds(origin * CHUNK, CHUNK), :].astype(jnp.bfloat16),
                (((1,), (0,)), ((), ())),
                preferred_element_type=jnp.float32,
            )

        out_ref[...] = acc.reshape(B, SQ, 512)

    return pl.pallas_call(
        body,
        out_shape=jax.ShapeDtypeStruct((B, SQ, 512), jnp.float32),
        in_specs=[pl.BlockSpec(memory_space=pltpu.VMEM)] * 5,
        out_specs=pl.BlockSpec(memory_space=pltpu.VMEM),
        scratch_shapes=[
            pltpu.VMEM((2, ROWS, CHUNK), jnp.bfloat16),
            pltpu.SemaphoreType.DMA((2,)),
            pltpu.SemaphoreType.DMA((2,)),
        ],
        compiler_params=pltpu.CompilerParams(collective_id=0),
    )(x, Wq, K_ext, V_ext, Wo)


# device time: 19512 ns/iter; 1.1730x vs baseline; 1.1730x over previous
import functools

import jax
import jax.numpy as jnp
from jax import lax
from jax.experimental import pallas as pl
from jax.experimental.pallas import tpu as pltpu

N_DEV = 4
B, SQ, SKV, HQ_TOTAL, DH = 2, 128, 128, 16, 64
H_LOC = HQ_TOTAL // N_DEV
CHUNK = H_LOC * DH
ROWS = B * SQ


def kernel(x, Wq, K_ext, V_ext, Wo):
    def body(x_ref, wq_ref, k_ref, v_ref, wo_ref, out_ref,
             comm_ref, send_sems, recv_sems):
        my = lax.axis_index("i")
        peers = [lax.rem(my + d, N_DEV) for d in (1, 2, 3)]

        barrier_sem = pltpu.get_barrier_semaphore()
        for p in peers:
            pl.semaphore_signal(
                barrier_sem, inc=1,
                device_id=(p,), device_id_type=pl.DeviceIdType.MESH,
            )
        pl.semaphore_wait(barrier_sem, 3)

        x2d = x_ref[...].reshape(ROWS, 512).astype(jnp.bfloat16)
        wq_slice = wq_ref[:, pl.ds(my * CHUNK, CHUNK)].astype(jnp.bfloat16)
        q2d = lax.dot_general(
            x2d, wq_slice, (((1,), (0,)), ((), ())),
            preferred_element_type=jnp.float32,
        ).astype(jnp.bfloat16)

        for b in range(B):
            for h in range(H_LOC):
                qb = q2d[b * SQ:(b + 1) * SQ, h * DH:(h + 1) * DH]
                kb = k_ref[b, :, h, :].astype(jnp.bfloat16)
                vb = v_ref[b, :, h, :].astype(jnp.bfloat16)
                scores = lax.dot_general(
                    qb, kb, (((1,), (1,)), ((), ())),
                    preferred_element_type=jnp.float32,
                ) * 0.125
                m = jnp.max(scores, axis=-1, keepdims=True)
                w = jnp.exp(scores - m)
                w = w / jnp.sum(w, axis=-1, keepdims=True)
                ctx = lax.dot_general(
                    w.astype(jnp.bfloat16), vb, (((1,), (0,)), ((), ())),
                    preferred_element_type=jnp.float32,
                )
                comm_ref[0, b * SQ:(b + 1) * SQ, h * DH:(h + 1) * DH] = (
                    ctx.astype(jnp.bfloat16)
                )

        rdmas = []
        for d in (1, 2, 3):
            rdma = pltpu.make_async_remote_copy(
                src_ref=comm_ref.at[0],
                dst_ref=comm_ref.at[d],
                send_sem=send_sems.at[d - 1],
                recv_sem=recv_sems.at[d - 1],
                device_id=(peers[d - 1],),
                device_id_type=pl.DeviceIdType.MESH,
            )
            rdma.start()
            rdmas.append(rdma)

        acc = lax.dot_general(
            comm_ref[0], wo_ref[pl.ds(my * CHUNK, CHUNK), :].astype(jnp.bfloat16),
            (((1,), (0,)), ((), ())),
            preferred_element_type=jnp.float32,
        )

        for d in (1, 3, 2):
            rdmas[d - 1].wait_recv()
            origin = lax.rem(my + N_DEV - d, N_DEV)
            acc += lax.dot_general(
                comm_ref[d],
                wo_ref[pl.ds(origin * CHUNK, CHUNK), :].astype(jnp.bfloat16),
                (((1,), (0,)), ((), ())),
                preferred_element_type=jnp.float32,
            )

        out_ref[...] = acc.reshape(B, SQ, 512)

        for r in rdmas:
            r.wait_send()

        @functools.partial(pl.run_scoped, exit_sem=pltpu.SemaphoreType.REGULAR)
        def _(exit_sem):
            for p in peers:
                pl.semaphore_signal(
                    exit_sem, inc=1,
                    device_id=(p,), device_id_type=pl.DeviceIdType.MESH,
                )
            pl.semaphore_wait(exit_sem, 3)

    return pl.pallas_call(
        body,
        out_shape=jax.ShapeDtypeStruct((B, SQ, 512), jnp.float32),
        in_specs=[pl.BlockSpec(memory_space=pltpu.VMEM)] * 5,
        out_specs=pl.BlockSpec(memory_space=pltpu.VMEM),
        scratch_shapes=[
            pltpu.VMEM((4, ROWS, CHUNK), jnp.bfloat16),
            pltpu.SemaphoreType.DMA((3,)),
            pltpu.SemaphoreType.DMA((3,)),
        ],
        compiler_params=pltpu.CompilerParams(collective_id=0),
    )(x, Wq, K_ext, V_ext, Wo)
